# baseline (speedup 1.0000x reference)
import functools

import numpy as np
import jax
import jax.numpy as jnp
from jax import lax
from jax.experimental import pallas as pl
from jax.experimental.pallas import tpu as pltpu
from jax.experimental.pallas import tpu_sc as plsc

N_NODES = 10000
N_EDGES = 320000
D_MODEL = 128
N_HEAD = 8
DIM_HEAD = 16

L = 16
NC = 2
NS = 16
NW = NC * NS
EDGES_PER_W = N_EDGES // NW
CHUNK = 40
NCHUNK = EDGES_PER_W // CHUNK
N_PAD = 10112
ROWS_PER_TILE = N_PAD // NS
ACC_W = 136
C_OFF = float(np.exp(5.0)) / 2.0

BLK = 1000


def _folded_colmap() -> np.ndarray:
    m = np.zeros(D_MODEL, dtype=np.int32)
    for f in range(D_MODEL):
        j, p = f // L, f % L
        if p < N_HEAD:
            h, d = p, 2 * j
        else:
            h, d = 15 - p, 2 * j + 1
        m[f] = h * DIM_HEAD + d
    return m


_COLMAP = _folded_colmap()


def _num_selector() -> np.ndarray:
    s = np.zeros((ACC_W, D_MODEL), dtype=np.float32)
    for c in range(D_MODEL):
        s[c, c] = 1.0
    return s


def _den_selector() -> np.ndarray:
    s = np.zeros((ACC_W, D_MODEL), dtype=np.float32)
    for k in range(N_HEAD):
        h = 7 - k
        for d in range(DIM_HEAD):
            s[D_MODEL + k, h * DIM_HEAD + d] = 1.0
    return s


_S_NUM = _num_selector()
_S_DEN = _den_selector()


def _proj_body(h_ref, wq_ref, wk_ref, wv_ref, qf_ref, kv_ref, v_ref):
    hb = h_ref[...]
    qf_ref[...] = jnp.dot(hb, wq_ref[...], preferred_element_type=jnp.float32)
    kv_ref[:, :D_MODEL] = jnp.dot(hb, wk_ref[...],
                                  preferred_element_type=jnp.float32)
    vv = jnp.dot(hb, wv_ref[...], preferred_element_type=jnp.float32)
    kv_ref[:, D_MODEL:] = vv
    v_ref[...] = vv


_proj = pl.pallas_call(
    _proj_body,
    grid=(N_NODES // BLK,),
    in_specs=[
        pl.BlockSpec((BLK, D_MODEL), lambda i: (i, 0)),
        pl.BlockSpec((D_MODEL, D_MODEL), lambda i: (0, 0)),
        pl.BlockSpec((D_MODEL, D_MODEL), lambda i: (0, 0)),
        pl.BlockSpec((D_MODEL, D_MODEL), lambda i: (0, 0)),
    ],
    out_specs=[
        pl.BlockSpec((BLK, D_MODEL), lambda i: (i, 0)),
        pl.BlockSpec((BLK, 2 * D_MODEL), lambda i: (i, 0)),
        pl.BlockSpec((BLK, D_MODEL), lambda i: (i, 0)),
    ],
    out_shape=[
        jax.ShapeDtypeStruct((N_NODES, D_MODEL), jnp.float32),
        jax.ShapeDtypeStruct((N_NODES, 2 * D_MODEL), jnp.float32),
        jax.ShapeDtypeStruct((N_NODES, D_MODEL), jnp.float32),
    ],
)


@functools.cache
def _build_edge_kernel():
  mesh = plsc.VectorSubcoreMesh(
      core_axis_name="c", subcore_axis_name="s", num_cores=NC, num_subcores=NS
  )

  @functools.partial(
    pl.kernel,
    out_type=jax.ShapeDtypeStruct((NC, N_PAD, ACC_W), jnp.float32),
    mesh=mesh,
    compiler_params=pltpu.CompilerParams(use_tc_tiling_on_sc=False),
    scratch_types=[
        pltpu.VMEM_SHARED((N_PAD, ACC_W), jnp.float32),
        pltpu.VMEM((2, CHUNK), jnp.int32),
        pltpu.VMEM((2, CHUNK), jnp.int32),
        pltpu.VMEM((CHUNK, 2 * D_MODEL), jnp.float32),
        pltpu.VMEM((CHUNK, 2 * D_MODEL), jnp.float32),
        pltpu.VMEM((CHUNK, D_MODEL), jnp.float32),
        pltpu.VMEM((CHUNK, D_MODEL), jnp.float32),
        pltpu.VMEM((CHUNK, ACC_W), jnp.float32),
        pltpu.VMEM((CHUNK, ACC_W), jnp.float32),
        pltpu.VMEM((CHUNK,), jnp.int32),
        pltpu.VMEM((CHUNK,), jnp.int32),
        pltpu.SemaphoreType.DMA,
        pltpu.SemaphoreType.DMA,
        pltpu.SemaphoreType.DMA,
        pltpu.SemaphoreType.DMA,
        pltpu.SemaphoreType.DMA,
        pltpu.SemaphoreType.DMA,
    ],
  )
  def _edge_kernel(kv_hbm, qf_hbm, ei_hbm, out_hbm,
                   acc, idx_a, idx_b, kv_a, kv_b, q_a, q_b,
                   wv_a, wv_b, didx_a, didx_b,
                   semi_a, semi_b, semr_a, semr_b, sems_a, sems_b):
    cid = lax.axis_index("c")
    sid = lax.axis_index("s")
    r0 = sid * ROWS_PER_TILE
    zvec = jnp.zeros((L,), dtype=jnp.float32)

    @plsc.parallel_loop(0, CHUNK, step=1, unroll=8)
    def _zero_row(zi):
        for j in range(D_MODEL // L):
            wv_a[zi, pl.ds(j * L, L)] = zvec
        wv_a[zi, pl.ds(ACC_W - L, L)] = zvec

    for rep in range(ROWS_PER_TILE // CHUNK):
        pltpu.sync_copy(wv_a, acc.at[pl.ds(r0 + rep * CHUNK, CHUNK)])
    _TAIL = ROWS_PER_TILE - (ROWS_PER_TILE // CHUNK) * CHUNK
    pltpu.sync_copy(
        wv_a.at[pl.ds(0, _TAIL)],
        acc.at[pl.ds(r0 + ROWS_PER_TILE - _TAIL, _TAIL)])
    plsc.subcore_barrier()

    base = (cid * NS + sid) * EDGES_PER_W

    sets = (
        (idx_a, kv_a, q_a, wv_a, didx_a, semi_a, semr_a, sems_a),
        (idx_b, kv_b, q_b, wv_b, didx_b, semi_b, semr_b, sems_b),
    )

    def fire_idx(c, st):
        idx, _, _, _, _, semi, _, _ = st
        pltpu.async_copy(ei_hbm.at[:, pl.ds(base + c * CHUNK, CHUNK)], idx,
                         semi)

    def wait_idx(c, st):
        idx, _, _, _, _, semi, _, _ = st
        pltpu.make_async_copy(ei_hbm.at[:, pl.ds(base + c * CHUNK, CHUNK)],
                              idx, semi).wait()

    def fire_rows(st):
        idx, kvb, qb, _, _, _, semr, _ = st
        pltpu.async_copy(kv_hbm.at[idx.at[0]], kvb, semr)
        pltpu.async_copy(qf_hbm.at[idx.at[1]], qb, semr)

    def wait_rows(st):
        idx, kvb, qb, _, _, _, semr, _ = st
        pltpu.make_async_copy(kv_hbm.at[idx.at[0]], kvb, semr).wait()
        pltpu.make_async_copy(qf_hbm.at[idx.at[1]], qb, semr).wait()

    def copy_didx(st):
        idx, _, _, _, didx, _, _, _ = st
        didx[pl.ds(0, L)] = idx[1, pl.ds(0, L)]
        didx[pl.ds(L, L)] = idx[1, pl.ds(L, L)]
        didx[pl.ds(CHUNK - L, L)] = idx[1, pl.ds(CHUNK - L, L)]

    def compute(st):
        _, kvb, qb, wvb, _, _, _, _ = st

        @plsc.parallel_loop(0, CHUNK, step=1, unroll=10)
        def body(e):
            prods = [kvb[e, pl.ds(j * L, L)] * qb[e, pl.ds(j * L, L)]
                     for j in range(D_MODEL // L)]
            while len(prods) > 1:
                prods = [a + b for a, b in zip(prods[::2], prods[1::2])]
            t = prods[0]
            score = t + lax.rev(t, (0,))
            u = jnp.clip(score * 0.25, -5.0, 5.0)
            wvec = jnp.exp(jnp.exp(u) - C_OFF)
            wvb[e, pl.ds(120, L)] = wvec
            for hh in range(N_HEAD):
                wvb[e, pl.ds(hh * L, L)] = (
                    wvec[hh] * kvb[e, pl.ds(D_MODEL + hh * L, L)])

    def fire_scatter(st):
        _, _, _, wvb, didx, _, _, sems = st
        pltpu.make_async_copy(wvb, acc.at[didx], sems).start(add=True)

    def wait_scatter(st):
        _, _, _, wvb, didx, _, _, sems = st
        pltpu.make_async_copy(wvb, acc.at[didx], sems).wait()

    fire_idx(0, sets[0])
    wait_idx(0, sets[0])
    fire_rows(sets[0])
    fire_idx(1, sets[1])
    wait_idx(1, sets[1])
    fire_rows(sets[1])

    NPAIR = NCHUNK // 2

    def half(i, c_next, st):
        @pl.when(i > 0)
        def _():
            wait_scatter(st)

        wait_rows(st)
        copy_didx(st)

        @pl.when(i < NPAIR - 1)
        def _():
            fire_idx(c_next, st)

        compute(st)
        fire_scatter(st)

        @pl.when(i < NPAIR - 1)
        def _():
            wait_idx(c_next, st)
            fire_rows(st)

    def pair_body(i, carry):
        c = 2 * i
        half(i, c + 2, sets[0])
        half(i, c + 3, sets[1])
        return carry

    lax.fori_loop(0, NPAIR, pair_body, 0)
    wait_scatter(sets[0])
    wait_scatter(sets[1])
    plsc.subcore_barrier()
    pltpu.sync_copy(acc.at[pl.ds(r0, ROWS_PER_TILE)],
                    out_hbm.at[cid, pl.ds(r0, ROWS_PER_TILE)])

  return _edge_kernel


def _combine_body(p_ref, v_ref, sn_ref, sd_ref, o_ref):
    p = p_ref[0] + p_ref[1]
    num = jnp.dot(p, sn_ref[...], preferred_element_type=jnp.float32)
    den = jnp.dot(p, sd_ref[...], preferred_element_type=jnp.float32)
    o_ref[...] = jnp.where(den > 0.0, num / den, v_ref[...])


_combine = pl.pallas_call(
    _combine_body,
    grid=(N_NODES // BLK,),
    in_specs=[
        pl.BlockSpec((NC, BLK, ACC_W), lambda i: (0, i, 0)),
        pl.BlockSpec((BLK, D_MODEL), lambda i: (i, 0)),
        pl.BlockSpec((ACC_W, D_MODEL), lambda i: (0, 0)),
        pl.BlockSpec((ACC_W, D_MODEL), lambda i: (0, 0)),
    ],
    out_specs=pl.BlockSpec((BLK, D_MODEL), lambda i: (i, 0)),
    out_shape=jax.ShapeDtypeStruct((N_NODES, D_MODEL), jnp.float32),
)


def kernel(h, edge_index, Wq, Wk, Wv):
    colmap = jnp.asarray(_COLMAP)
    qf, kv, v = _proj(h, Wq[:, colmap], Wk[:, colmap], Wv)
    ei = edge_index.astype(jnp.int32)
    partials = _build_edge_kernel()(kv, qf, ei)
    out = _combine(partials, v, jnp.asarray(_S_NUM), jnp.asarray(_S_DEN))
    return out.reshape(N_NODES, N_HEAD, DIM_HEAD)


# speedup vs baseline: 1.0306x; 1.0306x over previous
import functools

import numpy as np
import jax
import jax.numpy as jnp
from jax import lax
from jax.experimental import pallas as pl
from jax.experimental.pallas import tpu as pltpu
from jax.experimental.pallas import tpu_sc as plsc

N_NODES = 10000
N_EDGES = 320000
D_MODEL = 128
N_HEAD = 8
DIM_HEAD = 16

L = 16
NC = 2
NS = 16
NW = NC * NS
EDGES_PER_W = N_EDGES // NW
CHUNK = 40
NCHUNK = EDGES_PER_W // CHUNK
N_PAD = 10112
ROWS_PER_TILE = N_PAD // NS
ACC_W = 136
C_OFF = float(np.exp(5.0)) / 2.0

BLK = 1000


def _folded_colmap() -> np.ndarray:
    m = np.zeros(D_MODEL, dtype=np.int32)
    for f in range(D_MODEL):
        j, p = f // L, f % L
        if p < N_HEAD:
            h, d = p, 2 * j
        else:
            h, d = 15 - p, 2 * j + 1
        m[f] = h * DIM_HEAD + d
    return m


_COLMAP = _folded_colmap()


def _num_selector() -> np.ndarray:
    s = np.zeros((ACC_W, D_MODEL), dtype=np.float32)
    for c in range(D_MODEL):
        s[c, c] = 1.0
    return s


def _den_selector() -> np.ndarray:
    s = np.zeros((ACC_W, D_MODEL), dtype=np.float32)
    for k in range(N_HEAD):
        h = 7 - k
        for d in range(DIM_HEAD):
            s[D_MODEL + k, h * DIM_HEAD + d] = 1.0
    return s


_S_NUM = _num_selector()
_S_DEN = _den_selector()


def _proj_body(h_ref, wq_ref, wk_ref, wv_ref, qf_ref, kf_ref, v_ref):
    hb = h_ref[...]
    qf_ref[...] = jnp.dot(hb, wq_ref[...], preferred_element_type=jnp.float32)
    kf_ref[...] = jnp.dot(hb, wk_ref[...], preferred_element_type=jnp.float32)
    v_ref[...] = jnp.dot(hb, wv_ref[...], preferred_element_type=jnp.float32)


_proj = pl.pallas_call(
    _proj_body,
    grid=(N_NODES // BLK,),
    in_specs=[
        pl.BlockSpec((BLK, D_MODEL), lambda i: (i, 0)),
        pl.BlockSpec((D_MODEL, D_MODEL), lambda i: (0, 0)),
        pl.BlockSpec((D_MODEL, D_MODEL), lambda i: (0, 0)),
        pl.BlockSpec((D_MODEL, D_MODEL), lambda i: (0, 0)),
    ],
    out_specs=[
        pl.BlockSpec((BLK, D_MODEL), lambda i: (i, 0)),
        pl.BlockSpec((BLK, D_MODEL), lambda i: (i, 0)),
        pl.BlockSpec((BLK, D_MODEL), lambda i: (i, 0)),
    ],
    out_shape=[jax.ShapeDtypeStruct((N_NODES, D_MODEL), jnp.float32)] * 3,
)


@functools.cache
def _build_edge_kernel():
  mesh = plsc.VectorSubcoreMesh(
      core_axis_name="c", subcore_axis_name="s", num_cores=NC, num_subcores=NS
  )

  @functools.partial(
    pl.kernel,
    out_type=jax.ShapeDtypeStruct((NC, N_PAD, ACC_W), jnp.float32),
    mesh=mesh,
    compiler_params=pltpu.CompilerParams(use_tc_tiling_on_sc=False),
    scratch_types=[
        pltpu.VMEM_SHARED((N_PAD, ACC_W), jnp.float32),
        pltpu.VMEM((2, CHUNK), jnp.int32),
        pltpu.VMEM((2, CHUNK), jnp.int32),
        pltpu.VMEM((CHUNK, D_MODEL), jnp.float32),
        pltpu.VMEM((CHUNK, D_MODEL), jnp.float32),
        pltpu.VMEM((CHUNK, D_MODEL), jnp.float32),
        pltpu.VMEM((CHUNK, D_MODEL), jnp.float32),
        pltpu.VMEM((CHUNK, D_MODEL), jnp.float32),
        pltpu.VMEM((CHUNK, D_MODEL), jnp.float32),
        pltpu.VMEM((CHUNK, ACC_W), jnp.float32),
        pltpu.VMEM((CHUNK, ACC_W), jnp.float32),
        pltpu.VMEM((CHUNK,), jnp.int32),
        pltpu.VMEM((CHUNK,), jnp.int32),
        pltpu.SemaphoreType.DMA,
        pltpu.SemaphoreType.DMA,
        pltpu.SemaphoreType.DMA,
        pltpu.SemaphoreType.DMA,
        pltpu.SemaphoreType.DMA,
        pltpu.SemaphoreType.DMA,
    ],
  )
  def _edge_kernel(kf_hbm, qf_hbm, v_hbm, ei_hbm, out_hbm,
                   acc, idx_a, idx_b, k_a, k_b, q_a, q_b, v_a, v_b,
                   wv_a, wv_b, didx_a, didx_b,
                   semi_a, semi_b, semr_a, semr_b, sems_a, sems_b):
    cid = lax.axis_index("c")
    sid = lax.axis_index("s")
    r0 = sid * ROWS_PER_TILE
    zvec = jnp.zeros((L,), dtype=jnp.float32)

    @plsc.parallel_loop(0, CHUNK, step=1, unroll=8)
    def _zero_row(zi):
        for j in range(D_MODEL // L):
            wv_a[zi, pl.ds(j * L, L)] = zvec
        wv_a[zi, pl.ds(ACC_W - L, L)] = zvec

    for rep in range(ROWS_PER_TILE // CHUNK):
        pltpu.sync_copy(wv_a, acc.at[pl.ds(r0 + rep * CHUNK, CHUNK)])
    _TAIL = ROWS_PER_TILE - (ROWS_PER_TILE // CHUNK) * CHUNK
    pltpu.sync_copy(
        wv_a.at[pl.ds(0, _TAIL)],
        acc.at[pl.ds(r0 + ROWS_PER_TILE - _TAIL, _TAIL)])
    plsc.subcore_barrier()

    base = (cid * NS + sid) * EDGES_PER_W

    sets = (
        (idx_a, k_a, q_a, v_a, wv_a, didx_a, semi_a, semr_a, sems_a),
        (idx_b, k_b, q_b, v_b, wv_b, didx_b, semi_b, semr_b, sems_b),
    )

    def fire_idx(c, st):
        idx, _, _, _, _, _, semi, _, _ = st
        pltpu.async_copy(ei_hbm.at[:, pl.ds(base + c * CHUNK, CHUNK)], idx,
                         semi)

    def wait_idx(c, st):
        idx, _, _, _, _, _, semi, _, _ = st
        pltpu.make_async_copy(ei_hbm.at[:, pl.ds(base + c * CHUNK, CHUNK)],
                              idx, semi).wait()

    def fire_rows(st):
        idx, kb, qb, vb, _, _, _, semr, _ = st
        pltpu.async_copy(kf_hbm.at[idx.at[0]], kb, semr)
        pltpu.async_copy(qf_hbm.at[idx.at[1]], qb, semr)
        pltpu.async_copy(v_hbm.at[idx.at[0]], vb, semr)

    def wait_rows(st):
        idx, kb, qb, vb, _, _, _, semr, _ = st
        pltpu.make_async_copy(kf_hbm.at[idx.at[0]], kb, semr).wait()
        pltpu.make_async_copy(qf_hbm.at[idx.at[1]], qb, semr).wait()
        pltpu.make_async_copy(v_hbm.at[idx.at[0]], vb, semr).wait()

    def copy_didx(st):
        idx, _, _, _, _, didx, _, _, _ = st
        didx[pl.ds(0, L)] = idx[1, pl.ds(0, L)]
        didx[pl.ds(L, L)] = idx[1, pl.ds(L, L)]
        didx[pl.ds(CHUNK - L, L)] = idx[1, pl.ds(CHUNK - L, L)]

    def compute(st):
        _, kb, qb, vb, wvb, _, _, _, _ = st

        @plsc.parallel_loop(0, CHUNK, step=1, unroll=10)
        def body(e):
            prods = [kb[e, pl.ds(j * L, L)] * qb[e, pl.ds(j * L, L)]
                     for j in range(D_MODEL // L)]
            while len(prods) > 1:
                prods = [a + b for a, b in zip(prods[::2], prods[1::2])]
            t = prods[0]
            score = t + lax.rev(t, (0,))
            u = jnp.clip(score * 0.25, -5.0, 5.0)
            wvec = jnp.exp(jnp.exp(u) - C_OFF)
            wvb[e, pl.ds(120, L)] = wvec
            for hh in range(N_HEAD):
                wvb[e, pl.ds(hh * L, L)] = wvec[hh] * vb[e, pl.ds(hh * L, L)]

    def fire_scatter(st):
        _, _, _, _, wvb, didx, _, _, sems = st
        pltpu.make_async_copy(wvb, acc.at[didx], sems).start(add=True)

    def wait_scatter(st):
        _, _, _, _, wvb, didx, _, _, sems = st
        pltpu.make_async_copy(wvb, acc.at[didx], sems).wait()

    fire_idx(0, sets[0])
    wait_idx(0, sets[0])
    fire_rows(sets[0])
    fire_idx(1, sets[1])
    wait_idx(1, sets[1])
    fire_rows(sets[1])

    NPAIR = NCHUNK // 2

    def half(i, c_next, st):
        @pl.when(i > 0)
        def _():
            wait_scatter(st)

        wait_rows(st)
        copy_didx(st)

        @pl.when(i < NPAIR - 1)
        def _():
            fire_idx(c_next, st)

        compute(st)
        fire_scatter(st)

        @pl.when(i < NPAIR - 1)
        def _():
            wait_idx(c_next, st)
            fire_rows(st)

    def pair_body(i, carry):
        c = 2 * i
        half(i, c + 2, sets[0])
        half(i, c + 3, sets[1])
        return carry

    lax.fori_loop(0, NPAIR, pair_body, 0)
    wait_scatter(sets[0])
    wait_scatter(sets[1])
    plsc.subcore_barrier()
    pltpu.sync_copy(acc.at[pl.ds(r0, ROWS_PER_TILE)],
                    out_hbm.at[cid, pl.ds(r0, ROWS_PER_TILE)])

  return _edge_kernel


def _combine_body(p_ref, v_ref, sn_ref, sd_ref, o_ref):
    p = p_ref[0] + p_ref[1]
    num = jnp.dot(p, sn_ref[...], preferred_element_type=jnp.float32)
    den = jnp.dot(p, sd_ref[...], preferred_element_type=jnp.float32)
    o_ref[...] = jnp.where(den > 0.0, num / den, v_ref[...])


_combine = pl.pallas_call(
    _combine_body,
    grid=(N_NODES // BLK,),
    in_specs=[
        pl.BlockSpec((NC, BLK, ACC_W), lambda i: (0, i, 0)),
        pl.BlockSpec((BLK, D_MODEL), lambda i: (i, 0)),
        pl.BlockSpec((ACC_W, D_MODEL), lambda i: (0, 0)),
        pl.BlockSpec((ACC_W, D_MODEL), lambda i: (0, 0)),
    ],
    out_specs=pl.BlockSpec((BLK, D_MODEL), lambda i: (i, 0)),
    out_shape=jax.ShapeDtypeStruct((N_NODES, D_MODEL), jnp.float32),
)


def kernel(h, edge_index, Wq, Wk, Wv):
    colmap = jnp.asarray(_COLMAP)
    qf, kf, v = _proj(h, Wq[:, colmap], Wk[:, colmap], Wv)
    ei = edge_index.astype(jnp.int32)
    partials = _build_edge_kernel()(kf, qf, v, ei)
    out = _combine(partials, v, jnp.asarray(_S_NUM), jnp.asarray(_S_DEN))
    return out.reshape(N_NODES, N_HEAD, DIM_HEAD)
